# baseline (speedup 1.0000x reference)
import functools

import jax
import jax.numpy as jnp
from jax import lax
from jax.experimental import pallas as pl
from jax.experimental.pallas import tpu as pltpu
from jax.experimental.pallas import tpu_sc as plsc

_N = 10000
_E = 320000
_FIN = 128
_HID = 64
_C = 40
_K = 10
_FP = 64
_L = 16
_NC = 2
_NS = 16
_NW = _NC * _NS
_EW = _E // _NW
_NPAD = 10240
_CH = 6400
_NCHUNK = _E // _CH
_GRP = _CH // _L


def _mlp_body(x_ref, w1_ref, b1_ref, w2_ref, b2_ref, o_ref):
    h1 = jnp.dot(x_ref[...], w1_ref[...], preferred_element_type=jnp.float32)
    h1 = jnp.maximum(h1 + b1_ref[...], 0.0)
    o_ref[...] = (
        jnp.dot(h1, w2_ref[...], preferred_element_type=jnp.float32)
        + b2_ref[...]
    )


def _mlp(x, W1, b1, W2p, b2p):
    rb = 2000
    return pl.pallas_call(
        _mlp_body,
        grid=(_N // rb,),
        in_specs=[
            pl.BlockSpec((rb, _FIN), lambda i: (i, 0)),
            pl.BlockSpec((_FIN, _HID), lambda i: (0, 0)),
            pl.BlockSpec((1, _HID), lambda i: (0, 0)),
            pl.BlockSpec((_HID, _FP), lambda i: (0, 0)),
            pl.BlockSpec((1, _FP), lambda i: (0, 0)),
        ],
        out_specs=pl.BlockSpec((rb, _FP), lambda i: (i, 0)),
        out_shape=jax.ShapeDtypeStruct((_N, _FP), jnp.float32),
    )(x, W1, b1, W2p, b2p)


_MESH = plsc.VectorSubcoreMesh(core_axis_name="c", subcore_axis_name="s")
_SC_PARAMS = pltpu.CompilerParams(needs_layout_passes=False)


@functools.partial(
    pl.kernel,
    out_type=[
        jax.ShapeDtypeStruct((_E,), jnp.int32),
        jax.ShapeDtypeStruct((_NW, _NPAD), jnp.float32),
    ],
    mesh=_MESH,
    scratch_types=[
        pltpu.VMEM((_EW,), jnp.int32),
        pltpu.VMEM((_EW,), jnp.int32),
        pltpu.VMEM((_EW,), jnp.int32),
        pltpu.VMEM((_NPAD,), jnp.float32),
        pltpu.SemaphoreType.DMA,
    ],
    compiler_params=_SC_PARAMS,
)
def _preprocess(row_hbm, col_hbm, rc_hbm, degp_hbm, row_v, col_v, rc_v,
                deg_v, sem):
    cid = lax.axis_index("c")
    sid = lax.axis_index("s")
    wid = cid * _NS + sid
    base = wid * _EW

    pltpu.async_copy(row_hbm.at[pl.ds(base, _EW)], row_v, sem).wait()
    pltpu.async_copy(col_hbm.at[pl.ds(base, _EW)], col_v, sem).wait()

    @plsc.parallel_loop(0, _NPAD // _L, unroll=8)
    def _zero(i):
        deg_v[pl.ds(i * _L, _L)] = jnp.zeros((_L,), jnp.float32)

    ones = jnp.ones((_L,), jnp.float32)

    @plsc.parallel_loop(0, _EW // _L, unroll=5)
    def _edges(i):
        r = row_v[pl.ds(i * _L, _L)]
        c = col_v[pl.ds(i * _L, _L)]
        rc_v[pl.ds(i * _L, _L)] = jnp.bitwise_or(lax.shift_left(r, 14), c)
        plsc.addupdate_scatter(deg_v, [c], ones)

    pltpu.async_copy(rc_v, rc_hbm.at[pl.ds(base, _EW)], sem).wait()
    pltpu.async_copy(deg_v, degp_hbm.at[wid], sem).wait()


@functools.partial(
    pl.kernel,
    out_type=jax.ShapeDtypeStruct((_FP, _N), jnp.float32),
    mesh=_MESH,
    scratch_types=[
        pltpu.VMEM((_N,), jnp.float32),
        pltpu.VMEM((_N,), jnp.float32),
        pltpu.VMEM((_N,), jnp.float32),
        pltpu.VMEM((_N,), jnp.float32),
        pltpu.VMEM((_N,), jnp.float32),
        pltpu.VMEM((_N,), jnp.float32),
        pltpu.VMEM((_NPAD,), jnp.float32),
        pltpu.VMEM((_NPAD,), jnp.float32),
        pltpu.VMEM((_NPAD,), jnp.float32),
        pltpu.VMEM((_CH,), jnp.int32),
        pltpu.VMEM((_CH,), jnp.int32),
        pltpu.VMEM((_N,), jnp.int32),
        pltpu.VMEM((_K + 1, _L), jnp.float32),
        pltpu.SemaphoreType.DMA,
        pltpu.SemaphoreType.DMA,
    ],
    compiler_params=_SC_PARAMS,
)
def _propagate(hT_hbm, degp_hbm, rc_hbm, tempb_hbm, acc_hbm,
               u0, s0, a0, u1, s1, a1, dis_v, dp0, dp1, eb0, eb1, up, tv,
               sem0, sem1):
    cid = lax.axis_index("c")
    sid = lax.axis_index("s")
    wid = cid * _NS + sid
    f0 = 2 * wid
    f1 = f0 + 1

    pltpu.sync_copy(tempb_hbm, tv)

    @plsc.parallel_loop(0, _NPAD // _L, unroll=8)
    def _zero(i):
        dis_v[pl.ds(i * _L, _L)] = jnp.zeros((_L,), jnp.float32)

    pltpu.async_copy(degp_hbm.at[0], dp0, sem0)
    pltpu.async_copy(degp_hbm.at[1], dp1, sem1)

    def _deg_pair(j, _):
        nxt0 = 2 * j + 2
        nxt0 = jnp.where(nxt0 >= _NW, 0, nxt0)
        nxt1 = 2 * j + 3
        nxt1 = jnp.where(nxt1 >= _NW, 1, nxt1)

        pltpu.make_async_copy(degp_hbm.at[0], dp0, sem0).wait()

        @plsc.parallel_loop(0, _NPAD // _L, unroll=4)
        def _acc0(i):
            sl = pl.ds(i * _L, _L)
            dis_v[sl] = dis_v[sl] + dp0[sl]

        pltpu.async_copy(degp_hbm.at[nxt0], dp0, sem0)

        pltpu.make_async_copy(degp_hbm.at[1], dp1, sem1).wait()

        @plsc.parallel_loop(0, _NPAD // _L, unroll=4)
        def _acc1(i):
            sl = pl.ds(i * _L, _L)
            dis_v[sl] = dis_v[sl] + dp1[sl]

        pltpu.async_copy(degp_hbm.at[nxt1], dp1, sem1)
        return 0

    lax.fori_loop(0, _NW // 2, _deg_pair, 0)
    pltpu.make_async_copy(degp_hbm.at[0], dp0, sem0).wait()
    pltpu.make_async_copy(degp_hbm.at[1], dp1, sem1).wait()

    @plsc.parallel_loop(0, _N // _L, unroll=5)
    def _rsqrt(i):
        sl = pl.ds(i * _L, _L)
        d = dis_v[sl] + 1.0
        bits = plsc.bitcast(d, jnp.int32)
        seed = 0x5F3759DF - lax.shift_right_logical(bits, 1)
        y = plsc.bitcast(seed, jnp.float32)
        hd = 0.5 * d
        y = y * (1.5 - hd * y * y)
        y = y * (1.5 - hd * y * y)
        y = y * (1.5 - hd * y * y)
        dis_v[sl] = y

    pltpu.async_copy(hT_hbm.at[f0], s0, sem0).wait()
    pltpu.async_copy(hT_hbm.at[f1], s1, sem1).wait()
    t0 = tv[0]
    zeros = jnp.zeros((_L,), jnp.float32)

    @plsc.parallel_loop(0, _N // _L, unroll=5)
    def _init(i):
        sl = pl.ds(i * _L, _L)
        d = dis_v[sl]
        h0 = s0[sl]
        a0[sl] = t0 * h0
        un0 = d * h0
        u0[sl] = un0
        s0[sl] = zeros
        h1 = s1[sl]
        a1[sl] = t0 * h1
        un1 = d * h1
        u1[sl] = un1
        s1[sl] = zeros
        pr = plsc.pack(un0, un1, format=plsc.PackFormat.INTERLEAVED)
        up[sl] = plsc.bitcast(pr, jnp.int32)

    def _fetch(ebuf, i):
        rc = ebuf[pl.ds(i * _L, _L)]
        col = jnp.bitwise_and(rc, 0x3FFF)
        row = lax.shift_right_logical(rc, 14)
        pv = plsc.load_gather(up, [row])
        ab = plsc.bitcast(pv, jnp.bfloat16)
        v0, v1 = plsc.unpack(ab, format=plsc.PackFormat.INTERLEAVED)
        return col, v0, v1

    def _gather_scatter(ebuf):
        @plsc.parallel_loop(0, _GRP, unroll=8)
        def _grp(i):
            col, v0, v1 = _fetch(ebuf, i)
            plsc.addupdate_scatter(s0, [col], v0)
            plsc.addupdate_scatter(s1, [col], v1)

    pltpu.async_copy(rc_hbm.at[pl.ds(0, _CH)], eb0, sem0)
    pltpu.async_copy(rc_hbm.at[pl.ds(_CH, _CH)], eb1, sem1)

    for k in range(_K):
        def _chunk_pair(j, _):
            nxt0 = 2 * j + 2
            nxt0 = jnp.where(nxt0 >= _NCHUNK, 0, nxt0)
            nxt1 = 2 * j + 3
            nxt1 = jnp.where(nxt1 >= _NCHUNK, 1, nxt1)

            pltpu.make_async_copy(rc_hbm.at[pl.ds(0, _CH)], eb0, sem0).wait()
            _gather_scatter(eb0)
            pltpu.async_copy(rc_hbm.at[pl.ds(nxt0 * _CH, _CH)], eb0, sem0)

            pltpu.make_async_copy(rc_hbm.at[pl.ds(0, _CH)], eb1, sem1).wait()
            _gather_scatter(eb1)
            pltpu.async_copy(rc_hbm.at[pl.ds(nxt1 * _CH, _CH)], eb1, sem1)
            return 0

        lax.fori_loop(0, _NCHUNK // 2, _chunk_pair, 0)

        t = tv[k + 1]

        @plsc.parallel_loop(0, _N // _L, unroll=5)
        def _point(i):
            sl = pl.ds(i * _L, _L)
            d = dis_v[sl]
            hn0 = d * (s0[sl] + u0[sl])
            a0[sl] = a0[sl] + t * hn0
            un0 = d * hn0
            u0[sl] = un0
            s0[sl] = zeros
            hn1 = d * (s1[sl] + u1[sl])
            a1[sl] = a1[sl] + t * hn1
            un1 = d * hn1
            u1[sl] = un1
            s1[sl] = zeros
            pr = plsc.pack(un0, un1, format=plsc.PackFormat.INTERLEAVED)
            up[sl] = plsc.bitcast(pr, jnp.int32)

    pltpu.make_async_copy(rc_hbm.at[pl.ds(0, _CH)], eb0, sem0).wait()
    pltpu.make_async_copy(rc_hbm.at[pl.ds(0, _CH)], eb1, sem1).wait()

    pltpu.async_copy(a0, acc_hbm.at[f0], sem0).wait()
    pltpu.async_copy(a1, acc_hbm.at[f1], sem1).wait()


def kernel(x, edge_index, W1, b1, W2, b2, temp):
    W2p = jnp.pad(W2, ((0, 0), (0, _FP - _C)))
    b2p = jnp.pad(b2, (0, _FP - _C)).reshape(1, _FP)
    b1r = b1.reshape(1, _HID)

    h = _mlp(x, W1, b1r, W2p, b2p)
    rc, degp = _preprocess(edge_index[0], edge_index[1])
    hT = h.T
    tempb = jnp.broadcast_to(temp[:, None], (_K + 1, _L))
    accT = _propagate(hT, degp, rc, tempb)
    return accT[:_C].T


# speedup vs baseline: 1.1262x; 1.1262x over previous
import functools

import jax
import jax.numpy as jnp
from jax import lax
from jax.experimental import pallas as pl
from jax.experimental.pallas import tpu as pltpu
from jax.experimental.pallas import tpu_sc as plsc

_N = 10000
_E = 320000
_FIN = 128
_HID = 64
_C = 40
_K = 10
_FP = 64
_L = 16
_NC = 2
_NS = 16
_NW = _NC * _NS
_EW = _E // _NW
_NPAD = 10240
_H = 672
_ST = _H * _L
_OV = 10240
_WREC = _ST + _OV
_CH = _ST // 2
_GRP = _CH // _L
_NPR = _N + _L
_DUMMY = _N


def _mlp_body(x_ref, w1_ref, b1_ref, w2_ref, b2_ref, o_ref):
    h1 = jnp.dot(x_ref[...], w1_ref[...], preferred_element_type=jnp.float32)
    h1 = jnp.maximum(h1 + b1_ref[...], 0.0)
    o_ref[...] = (
        jnp.dot(h1, w2_ref[...], preferred_element_type=jnp.float32)
        + b2_ref[...]
    )


def _mlp(x, W1, b1, W2p, b2p):
    rb = 2000
    return pl.pallas_call(
        _mlp_body,
        grid=(_N // rb,),
        in_specs=[
            pl.BlockSpec((rb, _FIN), lambda i: (i, 0)),
            pl.BlockSpec((_FIN, _HID), lambda i: (0, 0)),
            pl.BlockSpec((1, _HID), lambda i: (0, 0)),
            pl.BlockSpec((_HID, _FP), lambda i: (0, 0)),
            pl.BlockSpec((1, _FP), lambda i: (0, 0)),
        ],
        out_specs=pl.BlockSpec((rb, _FP), lambda i: (i, 0)),
        out_shape=jax.ShapeDtypeStruct((_N, _FP), jnp.float32),
    )(x, W1, b1, W2p, b2p)


_MESH = plsc.VectorSubcoreMesh(core_axis_name="c", subcore_axis_name="s")
_SC_PARAMS = pltpu.CompilerParams(needs_layout_passes=False)


@functools.partial(
    pl.kernel,
    out_type=[
        jax.ShapeDtypeStruct((_NW * _WREC,), jnp.int32),
        jax.ShapeDtypeStruct((_NW * _L,), jnp.int32),
        jax.ShapeDtypeStruct((_NW, _NPAD), jnp.float32),
    ],
    mesh=_MESH,
    scratch_types=[
        pltpu.VMEM((_EW,), jnp.int32),
        pltpu.VMEM((_EW,), jnp.int32),
        pltpu.VMEM((_EW,), jnp.int32),
        pltpu.VMEM((_NPAD,), jnp.float32),
        pltpu.VMEM((_EW,), jnp.int32),
        pltpu.VMEM((_EW,), jnp.int32),
        pltpu.VMEM((_WREC,), jnp.int32),
        pltpu.VMEM((_L,), jnp.int32),
        pltpu.VMEM((_L,), jnp.int32),
        pltpu.SemaphoreType.DMA,
    ],
    compiler_params=_SC_PARAMS,
)
def _preprocess(row_hbm, col_hbm, st_hbm, ovc_hbm, degp_hbm,
                row_v, col_v, rc_v, deg_v, hist_v, off_v, st_v, ovb_v,
                cnt_v, sem):
    cid = lax.axis_index("c")
    sid = lax.axis_index("s")
    wid = cid * _NS + sid
    base = wid * _EW
    iota = lax.iota(jnp.int32, _L)

    pltpu.async_copy(row_hbm.at[pl.ds(base, _EW)], row_v, sem).wait()
    pltpu.async_copy(col_hbm.at[pl.ds(base, _EW)], col_v, sem).wait()

    zf = jnp.zeros((_L,), jnp.float32)
    zi = jnp.zeros((_L,), jnp.int32)
    onesf = jnp.ones((_L,), jnp.float32)
    onesi = jnp.ones((_L,), jnp.int32)

    @plsc.parallel_loop(0, _NPAD // _L, unroll=8)
    def _zerod(i):
        deg_v[pl.ds(i * _L, _L)] = zf

    @plsc.parallel_loop(0, _EW // _L, unroll=8)
    def _zeroh(i):
        hist_v[pl.ds(i * _L, _L)] = zi

    @plsc.parallel_loop(0, _EW // _L, unroll=4)
    def _pack(i):
        sl = pl.ds(i * _L, _L)
        r = row_v[sl]
        c = col_v[sl]
        rc_v[sl] = jnp.bitwise_or(lax.shift_left(r, 14), c)
        plsc.addupdate_scatter(deg_v, [c], onesf)
        b = jnp.bitwise_and(c, 15)
        plsc.addupdate_scatter(hist_v, [b + i * _L], onesi)

    def _scan(i, run):
        sl = pl.ds(i * _L, _L)
        off_v[sl] = run
        return run + hist_v[sl]

    cvec = lax.fori_loop(0, _EW // _L, _scan, zi)

    over = jnp.maximum(cvec - _H, 0)
    ovincl = plsc.cumsum(over)
    ovb_v[...] = ovincl - over + _ST
    t_ov = jnp.sum(over)
    cnt16 = jnp.bitwise_and(t_ov + 15, -16)
    cnt_v[...] = jnp.broadcast_to(cnt16, (_L,))

    neutral = _DUMMY + iota

    @plsc.parallel_loop(0, _H, unroll=8)
    def _pref(i):
        st_v[pl.ds(i * _L, _L)] = neutral

    st_v[pl.ds(_ST + t_ov, _L)] = neutral

    @plsc.parallel_loop(0, _EW // _L, unroll=2)
    def _stripe(i):
        sl = pl.ds(i * _L, _L)
        rc = rc_v[sl]
        b = jnp.bitwise_and(rc, 15)
        rank, _ = plsc.scan_count(b)
        off = plsc.load_gather(off_v, [b + i * _L])
        g = off + rank
        ovb = plsc.load_gather(ovb_v, [b])
        dest = jnp.where(g < _H, g * _L + b, ovb + (g - _H))
        plsc.store_scatter(st_v, [dest], rc)

    pltpu.async_copy(st_v, st_hbm.at[pl.ds(wid * _WREC, _WREC)], sem).wait()
    pltpu.async_copy(cnt_v, ovc_hbm.at[pl.ds(wid * _L, _L)], sem).wait()
    pltpu.async_copy(deg_v, degp_hbm.at[wid], sem).wait()


@functools.partial(
    pl.kernel,
    out_type=jax.ShapeDtypeStruct((_FP, _N), jnp.float32),
    mesh=_MESH,
    scratch_types=[
        pltpu.VMEM((_N,), jnp.float32),
        pltpu.VMEM((_NPR,), jnp.float32),
        pltpu.VMEM((_N,), jnp.float32),
        pltpu.VMEM((_N,), jnp.float32),
        pltpu.VMEM((_NPR,), jnp.float32),
        pltpu.VMEM((_N,), jnp.float32),
        pltpu.VMEM((_NPAD,), jnp.float32),
        pltpu.VMEM((_NPAD,), jnp.float32),
        pltpu.VMEM((_NPAD,), jnp.float32),
        pltpu.VMEM((_CH,), jnp.int32),
        pltpu.VMEM((_CH,), jnp.int32),
        pltpu.VMEM((_OV,), jnp.int32),
        pltpu.VMEM((_N,), jnp.int32),
        pltpu.VMEM((_NW * _L,), jnp.int32),
        pltpu.VMEM((_K + 1, _L), jnp.float32),
        pltpu.SemaphoreType.DMA,
        pltpu.SemaphoreType.DMA,
    ],
    compiler_params=_SC_PARAMS,
)
def _propagate(hT_hbm, degp_hbm, st_hbm, ovc_hbm, tempb_hbm, acc_hbm,
               u0, s0, a0, u1, s1, a1, dis_v, dp0, dp1, eb0, eb1, ov_v, up,
               cnt_s, tv, sem0, sem1):
    cid = lax.axis_index("c")
    sid = lax.axis_index("s")
    wid = cid * _NS + sid
    f0 = 2 * wid
    f1 = f0 + 1

    pltpu.sync_copy(tempb_hbm, tv)
    pltpu.sync_copy(ovc_hbm, cnt_s)

    @plsc.parallel_loop(0, _NPAD // _L, unroll=8)
    def _zero(i):
        dis_v[pl.ds(i * _L, _L)] = jnp.zeros((_L,), jnp.float32)

    pltpu.async_copy(degp_hbm.at[0], dp0, sem0)
    pltpu.async_copy(degp_hbm.at[1], dp1, sem1)

    def _deg_pair(j, _):
        nxt0 = 2 * j + 2
        nxt0 = jnp.where(nxt0 >= _NW, 0, nxt0)
        nxt1 = 2 * j + 3
        nxt1 = jnp.where(nxt1 >= _NW, 1, nxt1)

        pltpu.make_async_copy(degp_hbm.at[0], dp0, sem0).wait()

        @plsc.parallel_loop(0, _NPAD // _L, unroll=4)
        def _acc0(i):
            sl = pl.ds(i * _L, _L)
            dis_v[sl] = dis_v[sl] + dp0[sl]

        pltpu.async_copy(degp_hbm.at[nxt0], dp0, sem0)

        pltpu.make_async_copy(degp_hbm.at[1], dp1, sem1).wait()

        @plsc.parallel_loop(0, _NPAD // _L, unroll=4)
        def _acc1(i):
            sl = pl.ds(i * _L, _L)
            dis_v[sl] = dis_v[sl] + dp1[sl]

        pltpu.async_copy(degp_hbm.at[nxt1], dp1, sem1)
        return 0

    lax.fori_loop(0, _NW // 2, _deg_pair, 0)
    pltpu.make_async_copy(degp_hbm.at[0], dp0, sem0).wait()
    pltpu.make_async_copy(degp_hbm.at[1], dp1, sem1).wait()

    @plsc.parallel_loop(0, _NPAD // _L, unroll=5)
    def _rsqrt(i):
        sl = pl.ds(i * _L, _L)
        d = dis_v[sl] + 1.0
        bits = plsc.bitcast(d, jnp.int32)
        seed = 0x5F3759DF - lax.shift_right_logical(bits, 1)
        y = plsc.bitcast(seed, jnp.float32)
        hd = 0.5 * d
        y = y * (1.5 - hd * y * y)
        y = y * (1.5 - hd * y * y)
        y = y * (1.5 - hd * y * y)
        dis_v[sl] = y

    pltpu.async_copy(hT_hbm.at[f0], s0, sem0).wait()
    pltpu.async_copy(hT_hbm.at[f1], s1, sem1).wait()
    t0 = tv[0]
    zeros = jnp.zeros((_L,), jnp.float32)

    @plsc.parallel_loop(0, _N // _L, unroll=5)
    def _init(i):
        sl = pl.ds(i * _L, _L)
        d = dis_v[sl]
        h0 = s0[sl]
        a0[sl] = t0 * h0
        un0 = d * h0
        u0[sl] = un0
        s0[sl] = zeros
        h1 = s1[sl]
        a1[sl] = t0 * h1
        un1 = d * h1
        u1[sl] = un1
        s1[sl] = zeros
        pr = plsc.pack(un0, un1, format=plsc.PackFormat.INTERLEAVED)
        up[sl] = plsc.bitcast(pr, jnp.int32)

    def _one_group(ebuf, i):
        rc = ebuf[pl.ds(i * _L, _L)]
        col = jnp.bitwise_and(rc, 0x3FFF)
        row = lax.shift_right_logical(rc, 14)
        pv = plsc.load_gather(up, [row])
        ab = plsc.bitcast(pv, jnp.bfloat16)
        v0, v1 = plsc.unpack(ab, format=plsc.PackFormat.INTERLEAVED)
        plsc.addupdate_scatter(s0, [col], v0)
        plsc.addupdate_scatter(s1, [col], v1)

    def _edges(ebuf):
        @plsc.parallel_loop(0, _GRP, unroll=8)
        def _grp(i):
            _one_group(ebuf, i)

    pltpu.async_copy(st_hbm.at[pl.ds(0, _CH)], eb0, sem0)
    pltpu.async_copy(st_hbm.at[pl.ds(_CH, _CH)], eb1, sem1)

    for k in range(_K):
        def _chunk_pair(j, _):
            nxtw = jnp.where(j + 1 >= _NW, 0, j + 1)
            nxt = nxtw * _WREC

            pltpu.make_async_copy(st_hbm.at[pl.ds(0, _CH)], eb0, sem0).wait()
            _edges(eb0)
            pltpu.async_copy(st_hbm.at[pl.ds(nxt, _CH)], eb0, sem0)

            pltpu.make_async_copy(st_hbm.at[pl.ds(0, _CH)], eb1, sem1).wait()
            _edges(eb1)
            pltpu.async_copy(st_hbm.at[pl.ds(nxt + _CH, _CH)], eb1, sem1)
            return 0

        lax.fori_loop(0, _NW, _chunk_pair, 0)

        def _ov_worker(w, _):
            cnt16 = cnt_s[pl.ds(w * _L, _L)][0]

            @pl.when(cnt16 > 0)
            def _do():
                pltpu.async_copy(
                    st_hbm.at[pl.ds(w * _WREC + _ST, _OV)], ov_v, sem0
                ).wait()

                def _ovg(g, _):
                    _one_group(ov_v, g)
                    return 0

                lax.fori_loop(0, cnt16 // _L, _ovg, 0)

            return 0

        lax.fori_loop(0, _NW, _ov_worker, 0)

        t = tv[k + 1]

        @plsc.parallel_loop(0, _N // _L, unroll=5)
        def _point(i):
            sl = pl.ds(i * _L, _L)
            d = dis_v[sl]
            hn0 = d * (s0[sl] + u0[sl])
            a0[sl] = a0[sl] + t * hn0
            un0 = d * hn0
            u0[sl] = un0
            s0[sl] = zeros
            hn1 = d * (s1[sl] + u1[sl])
            a1[sl] = a1[sl] + t * hn1
            un1 = d * hn1
            u1[sl] = un1
            s1[sl] = zeros
            pr = plsc.pack(un0, un1, format=plsc.PackFormat.INTERLEAVED)
            up[sl] = plsc.bitcast(pr, jnp.int32)

    pltpu.make_async_copy(st_hbm.at[pl.ds(0, _CH)], eb0, sem0).wait()
    pltpu.make_async_copy(st_hbm.at[pl.ds(0, _CH)], eb1, sem1).wait()

    pltpu.async_copy(a0, acc_hbm.at[f0], sem0).wait()
    pltpu.async_copy(a1, acc_hbm.at[f1], sem1).wait()


def kernel(x, edge_index, W1, b1, W2, b2, temp):
    W2p = jnp.pad(W2, ((0, 0), (0, _FP - _C)))
    b2p = jnp.pad(b2, (0, _FP - _C)).reshape(1, _FP)
    b1r = b1.reshape(1, _HID)

    h = _mlp(x, W1, b1r, W2p, b2p)
    st, ovc, degp = _preprocess(edge_index[0], edge_index[1])
    hT = jnp.pad(h.T, ((0, 0), (0, _L)))
    tempb = jnp.broadcast_to(temp[:, None], (_K + 1, _L))
    accT = _propagate(hT, degp, st, ovc, tempb)
    return accT[:_C].T
